# baseline (speedup 1.0000x reference)
import functools

import jax
import jax.numpy as jnp
from jax import lax
from jax.experimental import pallas as pl
from jax.experimental.pallas import tpu as pltpu
from jax.experimental.pallas import tpu_sc as plsc

_NUM_WORKERS = 32
_LANES = 16


@functools.lru_cache(maxsize=None)
def _make_emb(total, D, S):
    rows_per_w = total // _NUM_WORKERS
    CH = S
    chunks = rows_per_w // CH
    pairs = chunks // 2

    mesh = plsc.VectorSubcoreMesh(core_axis_name="c", subcore_axis_name="s")

    @functools.partial(
        pl.kernel,
        mesh=mesh,
        compiler_params=pltpu.CompilerParams(use_tc_tiling_on_sc=False),
        out_type=jax.ShapeDtypeStruct((total, 2 * D), jnp.float32),
        scratch_types=[
            pltpu.VMEM((S, D), jnp.float32),
            pltpu.VMEM((CH,), jnp.int32),
            pltpu.VMEM((CH,), jnp.int32),
            pltpu.VMEM((CH, D), jnp.float32),
            pltpu.VMEM((CH, D), jnp.float32),
            pltpu.VMEM((CH, 2 * D), jnp.float32),
            pltpu.VMEM((CH, 2 * D), jnp.float32),
            pltpu.SemaphoreType.DMA,
            pltpu.SemaphoreType.DMA,
            pltpu.SemaphoreType.DMA,
            pltpu.SemaphoreType.DMA,
            pltpu.SemaphoreType.DMA,
            pltpu.SemaphoreType.DMA,
        ],
    )
    def emb(idx_hbm, table_hbm, pos_hbm, out_hbm,
            pos_v, idx_a, idx_b, g_a, g_b, p_a, p_b,
            isem_a, isem_b, gsem_a, gsem_b, ssem_a, ssem_b):
        wid = lax.axis_index("s") * 2 + lax.axis_index("c")
        base = wid * rows_per_w

        idx_v = (idx_a, idx_b)
        g_v = (g_a, g_b)
        p_v = (p_a, p_b)
        isem = (isem_a, isem_b)
        gsem = (gsem_a, gsem_b)
        ssem = (ssem_a, ssem_b)

        def idx_load(c, k):
            pltpu.async_copy(idx_hbm.at[pl.ds(base + c * CH, CH)], idx_v[k], isem[k])

        def gather(k):
            pltpu.async_copy(table_hbm.at[idx_v[k]], g_v[k], gsem[k])

        def wait_idx(k):
            pltpu.make_async_copy(idx_hbm.at[pl.ds(0, CH)], idx_v[k], isem[k]).wait()

        def wait_gather(k):
            pltpu.make_async_copy(table_hbm.at[idx_v[k]], g_v[k], gsem[k]).wait()

        def wait_store(k):
            pltpu.make_async_copy(p_v[k], out_hbm.at[pl.ds(0, CH)], ssem[k]).wait()

        def step(c, k, first, last, load_next=True):
            if not last:
                wait_idx(1 - k)
                if not first:
                    pass
                gather(1 - k)
            wait_gather(k)
            if load_next:
                idx_load(c + 2, k)
            if not first:
                wait_store(k)

            def add_body(r, carry):
                for j in range(D // _LANES):
                    sl = pl.ds(j * _LANES, _LANES)
                    p_v[k][r, sl] = g_v[k][r, sl] + pos_v[r, sl]
                return carry

            lax.fori_loop(0, S, add_body, 0, unroll=2)
            pltpu.async_copy(p_v[k], out_hbm.at[pl.ds(base + c * CH, CH)], ssem[k])

        idx_load(0, 0)
        idx_load(1, 1)
        pltpu.sync_copy(pos_hbm, pos_v)
        wait_idx(0)
        gather(0)

        step(0, 0, True, False)
        step(1, 1, True, False)

        def body(i, carry):
            c = 2 + 2 * i
            step(c, 0, False, False)
            step(c + 1, 1, False, False)
            return carry

        lax.fori_loop(0, pairs - 2, body, 0)
        step(chunks - 2, 0, False, False, load_next=False)
        step(chunks - 1, 1, False, True, load_next=False)

        wait_store(0)
        wait_store(1)

    return emb


def kernel(inputs, word_table, pos_table):
    B, S = inputs.shape
    V, D = word_table.shape
    total = B * S
    idx_flat = inputs.reshape(total)
    emb = _make_emb(total, D, S)
    out = emb(idx_flat, word_table, pos_table)
    return out[:, :D].reshape(B, S, D)
